# baseline (speedup 1.0000x reference)
import jax
import jax.numpy as jnp
from jax import lax
from jax.experimental import pallas as pl
from jax.experimental.pallas import tpu as pltpu
from jax.experimental.pallas import tpu_sc as plsc

EDGES = 160000
ANGLES = 480000
FEAT = 128
FEAT_A = 16
NCRYS = 256
MIDF = 64
EPS = 1e-5
ISQRT2 = 1.0 / 2.0 ** 0.5

NSC = 2
NTILE = 16
NWORK = NSC * NTILE

G_PER_W = ANGLES // NWORK
G_WIN = 200
G_NWIN = G_PER_W // G_WIN
CZ_WIN = 2000

A_PER_W = ANGLES // NWORK
S_WIN = 1000
S_NWIN = A_PER_W // S_WIN
NCHUNK = 16
CW = FEAT // NCHUNK
E_PER_T = EDGES // NTILE
Z_WIN = 2000
Z_NWIN = E_PER_T // Z_WIN

BA = 1280
NBA = ANGLES // BA
BE = 1280
NBE = EDGES // BE

HIGH = lax.Precision.DEFAULT


def _gather_body(edge_hbm, i0_hbm, i1_hbm, g0_hbm, g1_hbm,
                 idx0, idx1, rows0, rows1, s00, s01, s10, s11):
    c = lax.axis_index("c")
    s = lax.axis_index("s")
    wid = s * NSC + c
    base = wid * G_PER_W
    sems0 = (s00, s01)
    sems1 = (s10, s11)

    def issue(w, b):
        off = base + w * G_WIN
        pltpu.sync_copy(i0_hbm.at[pl.ds(off, G_WIN)], idx0.at[b])
        pltpu.async_copy(edge_hbm.at[idx0.at[b]], rows0.at[b], sems0[b])
        pltpu.sync_copy(i1_hbm.at[pl.ds(off, G_WIN)], idx1.at[b])
        pltpu.async_copy(edge_hbm.at[idx1.at[b]], rows1.at[b], sems1[b])

    def drain_out(w, b):
        off = base + w * G_WIN
        pltpu.make_async_copy(edge_hbm.at[idx0.at[b]], rows0.at[b],
                              sems0[b]).wait()
        pltpu.sync_copy(rows0.at[b], g0_hbm.at[pl.ds(off, G_WIN)])
        pltpu.make_async_copy(edge_hbm.at[idx1.at[b]], rows1.at[b],
                              sems1[b]).wait()
        pltpu.sync_copy(rows1.at[b], g1_hbm.at[pl.ds(off, G_WIN)])

    issue(0, 0)

    def pair(w2, carry):
        w = 2 * w2

        @pl.when(w + 1 < G_NWIN)
        def _():
            issue(w + 1, 1)

        drain_out(w, 0)

        @pl.when(w + 2 < G_NWIN)
        def _():
            issue(w + 2, 0)

        @pl.when(w + 1 < G_NWIN)
        def _():
            drain_out(w + 1, 1)

        return carry

    lax.fori_loop(0, (G_NWIN + 1) // 2, pair, 0)


def _scatter_body(vals_hbm, sidx_hbm, ones_hbm, zeros_hbm,
                  out_sum, out_cnt, idx_all, vb2, zbuf, acc,
                  sem0, sem1):
    sc = lax.axis_index("c")
    t = lax.axis_index("s")
    wid = sc * NTILE + t
    abase = wid * A_PER_W
    ebase = t * E_PER_T
    sems = (sem0, sem1)

    pltpu.sync_copy(zeros_hbm, zbuf)

    def idx_load(w, carry):
        pltpu.sync_copy(sidx_hbm.at[pl.ds(abase + w * S_WIN, S_WIN)],
                        idx_all.at[w])
        return carry

    lax.fori_loop(0, S_NWIN, idx_load, 0)

    def zero_acc(j, carry):
        pltpu.sync_copy(zbuf, acc.at[pl.ds(ebase + j * Z_WIN, Z_WIN)])
        return carry

    lax.fori_loop(0, Z_NWIN, zero_acc, 0)
    plsc.subcore_barrier()

    pltpu.sync_copy(ones_hbm, vb2.at[0])

    def cnt_win(w, carry):
        pltpu.sync_copy(vb2.at[0], acc.at[idx_all.at[w]], add=True)
        return carry

    lax.fori_loop(0, S_NWIN, cnt_win, 0)
    plsc.subcore_barrier()

    def cnt_wb(j, carry):
        e0 = ebase + j * Z_WIN
        pltpu.sync_copy(acc.at[pl.ds(e0, Z_WIN)],
                        out_cnt.at[sc, pl.ds(e0, Z_WIN), pl.ds(0, CW)])
        pltpu.sync_copy(zbuf, acc.at[pl.ds(e0, Z_WIN)])
        return carry

    lax.fori_loop(0, Z_NWIN, cnt_wb, 0)
    plsc.subcore_barrier()

    for cix in range(NCHUNK):
        lanes = pl.ds(cix * CW, CW)

        def issue(w, b):
            pltpu.async_copy(
                vals_hbm.at[pl.ds(abase + w * S_WIN, S_WIN), lanes],
                vb2.at[b], sems[b])

        def drain_add(w, b):
            pltpu.make_async_copy(
                vals_hbm.at[pl.ds(abase, S_WIN), lanes],
                vb2.at[b], sems[b]).wait()
            pltpu.sync_copy(vb2.at[b], acc.at[idx_all.at[w]], add=True)

        issue(0, 0)

        def sum_pair(w2, carry2):
            w = 2 * w2 + 1

            @pl.when(w < S_NWIN)
            def _():
                issue(w, 1)

            drain_add(w - 1, 0)

            @pl.when(w + 1 < S_NWIN)
            def _():
                issue(w + 1, 0)

            @pl.when(w < S_NWIN)
            def _():
                drain_add(w, 1)

            return carry2

        lax.fori_loop(0, (S_NWIN + 1) // 2, sum_pair, 0)
        plsc.subcore_barrier()

        def sum_wb(j, carry2):
            e0 = ebase + j * Z_WIN
            pltpu.sync_copy(acc.at[pl.ds(e0, Z_WIN)],
                            out_sum.at[sc, pl.ds(e0, Z_WIN), lanes])
            pltpu.sync_copy(zbuf, acc.at[pl.ds(e0, Z_WIN)])
            return carry2

        lax.fori_loop(0, Z_NWIN, sum_wb, 0)
        plsc.subcore_barrier()


def _k2_body(angle_ref, g0_ref, g1_ref, idx_ref, w_ref, tg_ref, st_ref):
    i = pl.program_id(0)
    wa = w_ref[0:FEAT_A, :]
    w0 = w_ref[FEAT_A:FEAT_A + FEAT, :]
    w1 = w_ref[FEAT_A + FEAT:, :]
    dn = (((0,), (0,)), ((), ()))
    tg = (lax.dot_general(angle_ref[...], wa, dn,
                          preferred_element_type=jnp.float32,
                          precision=HIGH)
          + jnp.dot(g0_ref[...], w0, preferred_element_type=jnp.float32,
                    precision=HIGH)
          + jnp.dot(g1_ref[...], w1, preferred_element_type=jnp.float32,
                    precision=HIGH))
    tg_ref[...] = tg.astype(jnp.bfloat16)
    idxv = idx_ref[0]
    ot = (lax.broadcasted_iota(jnp.int32, (NCRYS, BA), 0) == idxv)
    otb = ot.astype(jnp.bfloat16)
    rhs = jnp.concatenate([tg, tg * tg], axis=1).astype(jnp.bfloat16)
    part = jnp.dot(otb, rhs, preferred_element_type=jnp.float32)
    cnt = jnp.sum(ot.astype(jnp.float32), axis=1, keepdims=True)
    part = jnp.concatenate(
        [part, jnp.broadcast_to(cnt, (NCRYS, FEAT))], axis=1)

    @pl.when(i == 0)
    def _():
        st_ref[...] = part

    @pl.when(i > 0)
    def _():
        st_ref[...] += part


def _norm_params(st, gamma, beta, width):
    ssum = st[:, :width]
    ssq = st[:, width:2 * width]
    cnt = jnp.maximum(st[:, 2 * width:2 * width + 1], 1.0)
    mean = ssum / cnt
    var = ssq / cnt - mean * mean
    scale = gamma * lax.rsqrt(var + EPS)
    shift = beta - mean * scale
    return scale, shift


def _k3_body(tg_ref, idx_ref, st_ref, g1_ref, b1_ref, wm_ref, vals_ref,
             sc_ref, sh_ref):
    i = pl.program_id(0)

    @pl.when(i == 0)
    def _():
        scale, shift = _norm_params(st_ref[...], g1_ref[...], b1_ref[...],
                                    2 * FEAT)
        sc_ref[...] = scale
        sh_ref[...] = shift

    idxv = idx_ref[0]
    ot = (lax.broadcasted_iota(jnp.int32, (NCRYS, BA), 0) == idxv)
    otb = ot.astype(jnp.bfloat16)
    dn = (((0,), (0,)), ((), ()))
    tab = jnp.concatenate(
        [sc_ref[...], sh_ref[...]], axis=1).astype(jnp.bfloat16)
    both = lax.dot_general(otb, tab, dn, preferred_element_type=jnp.float32)
    scale_r = both[:, :2 * FEAT]
    shift_r = both[:, 2 * FEAT:]
    normed = tg_ref[...].astype(jnp.float32) * scale_r + shift_r
    core = jnp.maximum(normed[:, :FEAT], 0.0)
    fh = normed[:, FEAT:]
    filt = jnp.tanh(jnp.sum(fh * wm_ref[...], axis=1, keepdims=True))
    vals_ref[...] = filt * core


def _k5_body(sum_ref, cnt_ref, idx_ref, summed_ref, st_ref):
    i = pl.program_id(0)
    s = sum_ref[0] + sum_ref[1]
    cnt = cnt_ref[0][:, 0:1] + cnt_ref[1][:, 0:1]
    summed = s / jnp.maximum(cnt, 1.0)
    summed_ref[...] = summed
    idxv = idx_ref[0]
    ot = (lax.broadcasted_iota(jnp.int32, (NCRYS, BE), 0) == idxv)
    otb = ot.astype(jnp.bfloat16)
    rhs = jnp.concatenate(
        [summed, summed * summed], axis=1).astype(jnp.bfloat16)
    part = jnp.dot(otb, rhs, preferred_element_type=jnp.float32)
    ccnt = jnp.sum(ot.astype(jnp.float32), axis=1, keepdims=True)
    part = jnp.concatenate(
        [part, jnp.broadcast_to(ccnt, (NCRYS, FEAT))], axis=1)

    @pl.when(i == 0)
    def _():
        st_ref[...] = part

    @pl.when(i > 0)
    def _():
        st_ref[...] += part


def _k6_body(summed_ref, idx_ref, st_ref, g2_ref, b2_ref, edge_ref,
             w10_ref, b10_ref, w20_ref, b20_ref,
             w11_ref, b11_ref, w21_ref, b21_ref,
             out_ref, sc_ref, sh_ref):
    i = pl.program_id(0)

    @pl.when(i == 0)
    def _():
        scale, shift = _norm_params(st_ref[...], g2_ref[...], b2_ref[...],
                                    FEAT)
        sc_ref[...] = scale
        sh_ref[...] = shift

    idxv = idx_ref[0]
    ot = (lax.broadcasted_iota(jnp.int32, (NCRYS, BE), 0) == idxv)
    otb = ot.astype(jnp.bfloat16)
    dn = (((0,), (0,)), ((), ()))
    tab = jnp.concatenate(
        [sc_ref[...], sh_ref[...]], axis=1).astype(jnp.bfloat16)
    both = lax.dot_general(otb, tab, dn, preferred_element_type=jnp.float32)
    scale_r = both[:, :FEAT]
    shift_r = both[:, FEAT:]
    s = summed_ref[...] * scale_r + shift_r
    h = jnp.maximum(jnp.dot(s, w10_ref[...], preferred_element_type=jnp.float32,
                            precision=HIGH) + b10_ref[...], 0.0)
    s = s + jnp.dot(h, w20_ref[...], preferred_element_type=jnp.float32,
                    precision=HIGH) + b20_ref[...]
    h = jnp.maximum(jnp.dot(s, w11_ref[...], preferred_element_type=jnp.float32,
                            precision=HIGH) + b11_ref[...], 0.0)
    s = s + jnp.dot(h, w21_ref[...], preferred_element_type=jnp.float32,
                    precision=HIGH) + b21_ref[...]
    out_ref[...] = ISQRT2 * jnp.maximum(edge_ref[...] + s, 0.0)


def kernel(edge, angle, angle_nbr_idx, crystal_edge_idx, crystal_angle_idx,
           W_full, W_mask, rW1_0, rb1_0, rW2_0, rb2_0, rW1_1, rb1_1, rW2_1,
           rb2_1, g1, b1, g2, b2):
    f32 = jnp.float32
    src_idx = angle_nbr_idx[:, 0].reshape(-1).astype(jnp.int32)
    nbr1_idx = angle_nbr_idx[:, 1].reshape(-1).astype(jnp.int32)
    ca3 = crystal_angle_idx.astype(jnp.int32).reshape(NBA, 1, BA)
    ce3 = crystal_edge_idx.astype(jnp.int32).reshape(NBE, 1, BE)

    k1 = pl.kernel(
        _gather_body,
        out_type=(
            jax.ShapeDtypeStruct((ANGLES, FEAT), f32),
            jax.ShapeDtypeStruct((ANGLES, FEAT), f32),
        ),
        mesh=plsc.VectorSubcoreMesh(core_axis_name="c", subcore_axis_name="s"),
        scratch_types=[
            pltpu.VMEM((2, G_WIN), jnp.int32),
            pltpu.VMEM((2, G_WIN), jnp.int32),
            pltpu.VMEM((2, G_WIN, FEAT), f32),
            pltpu.VMEM((2, G_WIN, FEAT), f32),
            pltpu.SemaphoreType.DMA,
            pltpu.SemaphoreType.DMA,
            pltpu.SemaphoreType.DMA,
            pltpu.SemaphoreType.DMA,
        ],
        compiler_params=pltpu.CompilerParams(use_tc_tiling_on_sc=False),
    )
    g0, g1_rows = k1(edge, src_idx, nbr1_idx)

    tg, stats1 = pl.pallas_call(
        _k2_body,
        grid=(NBA,),
        in_specs=[
            pl.BlockSpec((FEAT_A, BA), lambda i: (0, i)),
            pl.BlockSpec((BA, FEAT), lambda i: (i, 0)),
            pl.BlockSpec((BA, FEAT), lambda i: (i, 0)),
            pl.BlockSpec((1, 1, BA), lambda i: (i, 0, 0)),
            pl.BlockSpec((2 * FEAT + FEAT_A, 2 * FEAT), lambda i: (0, 0)),
        ],
        out_specs=[
            pl.BlockSpec((BA, 2 * FEAT), lambda i: (i, 0)),
            pl.BlockSpec((NCRYS, 5 * FEAT), lambda i: (0, 0)),
        ],
        out_shape=[
            jax.ShapeDtypeStruct((ANGLES, 2 * FEAT), jnp.bfloat16),
            jax.ShapeDtypeStruct((NCRYS, 5 * FEAT), f32),
        ],
    )(jnp.transpose(angle), g0, g1_rows, ca3, W_full)

    vals = pl.pallas_call(
        _k3_body,
        grid=(NBA,),
        in_specs=[
            pl.BlockSpec((BA, 2 * FEAT), lambda i: (i, 0)),
            pl.BlockSpec((1, 1, BA), lambda i: (i, 0, 0)),
            pl.BlockSpec((NCRYS, 5 * FEAT), lambda i: (0, 0)),
            pl.BlockSpec((1, 2 * FEAT), lambda i: (0, 0)),
            pl.BlockSpec((1, 2 * FEAT), lambda i: (0, 0)),
            pl.BlockSpec((1, FEAT), lambda i: (0, 0)),
        ],
        out_specs=pl.BlockSpec((BA, FEAT), lambda i: (i, 0)),
        out_shape=jax.ShapeDtypeStruct((ANGLES, FEAT), f32),
        scratch_shapes=[
            pltpu.VMEM((NCRYS, 2 * FEAT), f32),
            pltpu.VMEM((NCRYS, 2 * FEAT), f32),
        ],
    )(tg, ca3, stats1, g1.reshape(1, -1), b1.reshape(1, -1),
      W_mask.reshape(1, -1))

    ones_buf = jnp.ones((S_WIN, CW), f32)
    zeros_buf = jnp.zeros((Z_WIN, CW), f32)
    k4 = pl.kernel(
        _scatter_body,
        out_type=(
            jax.ShapeDtypeStruct((NSC, EDGES, FEAT), f32),
            jax.ShapeDtypeStruct((NSC, EDGES, FEAT), f32),
        ),
        mesh=plsc.VectorSubcoreMesh(core_axis_name="c", subcore_axis_name="s"),
        scratch_types=[
            pltpu.VMEM((S_NWIN, S_WIN), jnp.int32),
            pltpu.VMEM((2, S_WIN, CW), f32),
            pltpu.VMEM((Z_WIN, CW), f32),
            pltpu.VMEM_SHARED((EDGES, CW), f32),
            pltpu.SemaphoreType.DMA,
            pltpu.SemaphoreType.DMA,
        ],
        compiler_params=pltpu.CompilerParams(use_tc_tiling_on_sc=False),
    )
    sum2, cnt_parts = k4(vals, src_idx, ones_buf, zeros_buf)

    summed, stats2 = pl.pallas_call(
        _k5_body,
        grid=(NBE,),
        in_specs=[
            pl.BlockSpec((NSC, BE, FEAT), lambda i: (0, i, 0)),
            pl.BlockSpec((NSC, BE, FEAT), lambda i: (0, i, 0)),
            pl.BlockSpec((1, 1, BE), lambda i: (i, 0, 0)),
        ],
        out_specs=[
            pl.BlockSpec((BE, FEAT), lambda i: (i, 0)),
            pl.BlockSpec((NCRYS, 3 * FEAT), lambda i: (0, 0)),
        ],
        out_shape=[
            jax.ShapeDtypeStruct((EDGES, FEAT), f32),
            jax.ShapeDtypeStruct((NCRYS, 3 * FEAT), f32),
        ],
    )(sum2, cnt_parts, ce3)

    out = pl.pallas_call(
        _k6_body,
        grid=(NBE,),
        in_specs=[
            pl.BlockSpec((BE, FEAT), lambda i: (i, 0)),
            pl.BlockSpec((1, 1, BE), lambda i: (i, 0, 0)),
            pl.BlockSpec((NCRYS, 3 * FEAT), lambda i: (0, 0)),
            pl.BlockSpec((1, FEAT), lambda i: (0, 0)),
            pl.BlockSpec((1, FEAT), lambda i: (0, 0)),
            pl.BlockSpec((BE, FEAT), lambda i: (i, 0)),
            pl.BlockSpec((FEAT, MIDF), lambda i: (0, 0)),
            pl.BlockSpec((1, MIDF), lambda i: (0, 0)),
            pl.BlockSpec((MIDF, FEAT), lambda i: (0, 0)),
            pl.BlockSpec((1, FEAT), lambda i: (0, 0)),
            pl.BlockSpec((FEAT, MIDF), lambda i: (0, 0)),
            pl.BlockSpec((1, MIDF), lambda i: (0, 0)),
            pl.BlockSpec((MIDF, FEAT), lambda i: (0, 0)),
            pl.BlockSpec((1, FEAT), lambda i: (0, 0)),
        ],
        out_specs=pl.BlockSpec((BE, FEAT), lambda i: (i, 0)),
        out_shape=jax.ShapeDtypeStruct((EDGES, FEAT), f32),
        scratch_shapes=[
            pltpu.VMEM((NCRYS, FEAT), f32),
            pltpu.VMEM((NCRYS, FEAT), f32),
        ],
    )(summed, ce3, stats2, g2.reshape(1, -1), b2.reshape(1, -1), edge,
      rW1_0, rb1_0.reshape(1, -1), rW2_0, rb2_0.reshape(1, -1),
      rW1_1, rb1_1.reshape(1, -1), rW2_1, rb2_1.reshape(1, -1))
    return out


# speedup vs baseline: 1.0090x; 1.0090x over previous
import jax
import jax.numpy as jnp
from jax import lax
from jax.experimental import pallas as pl
from jax.experimental.pallas import tpu as pltpu
from jax.experimental.pallas import tpu_sc as plsc

EDGES = 160000
ANGLES = 480000
FEAT = 128
FEAT_A = 16
NCRYS = 256
MIDF = 64
EPS = 1e-5
ISQRT2 = 1.0 / 2.0 ** 0.5

NSC = 2
NTILE = 16
NWORK = NSC * NTILE

G_PER_W = ANGLES // NWORK
G_WIN = 200
G_NWIN = G_PER_W // G_WIN
CZ_WIN = 2000

A_PER_W = ANGLES // NWORK
S_WIN = 1000
S_NWIN = A_PER_W // S_WIN
NCHUNK = 16
CW = FEAT // NCHUNK
E_PER_T = EDGES // NTILE
Z_WIN = 1000
Z_NWIN = E_PER_T // Z_WIN

BA = 1280
NBA = ANGLES // BA
BE = 1280
NBE = EDGES // BE

HIGH = lax.Precision.DEFAULT


def _gather_body(edge_hbm, i0_hbm, i1_hbm, g0_hbm, g1_hbm,
                 idx0, idx1, rows0, rows1, s00, s01, s10, s11):
    c = lax.axis_index("c")
    s = lax.axis_index("s")
    wid = s * NSC + c
    base = wid * G_PER_W
    sems0 = (s00, s01)
    sems1 = (s10, s11)

    def issue(w, b):
        off = base + w * G_WIN
        pltpu.sync_copy(i0_hbm.at[pl.ds(off, G_WIN)], idx0.at[b])
        pltpu.async_copy(edge_hbm.at[idx0.at[b]], rows0.at[b], sems0[b])
        pltpu.sync_copy(i1_hbm.at[pl.ds(off, G_WIN)], idx1.at[b])
        pltpu.async_copy(edge_hbm.at[idx1.at[b]], rows1.at[b], sems1[b])

    def drain_out(w, b):
        off = base + w * G_WIN
        pltpu.make_async_copy(edge_hbm.at[idx0.at[b]], rows0.at[b],
                              sems0[b]).wait()
        pltpu.sync_copy(rows0.at[b], g0_hbm.at[pl.ds(off, G_WIN)])
        pltpu.make_async_copy(edge_hbm.at[idx1.at[b]], rows1.at[b],
                              sems1[b]).wait()
        pltpu.sync_copy(rows1.at[b], g1_hbm.at[pl.ds(off, G_WIN)])

    issue(0, 0)

    def pair(w2, carry):
        w = 2 * w2

        @pl.when(w + 1 < G_NWIN)
        def _():
            issue(w + 1, 1)

        drain_out(w, 0)

        @pl.when(w + 2 < G_NWIN)
        def _():
            issue(w + 2, 0)

        @pl.when(w + 1 < G_NWIN)
        def _():
            drain_out(w + 1, 1)

        return carry

    lax.fori_loop(0, (G_NWIN + 1) // 2, pair, 0)


def _scatter_body(vals_hbm, sidx_hbm, ones_hbm, zeros_hbm,
                  out_sum, out_cnt, idx_all, vb2, zbuf, acc,
                  sem0, sem1):
    sc = lax.axis_index("c")
    t = lax.axis_index("s")
    wid = sc * NTILE + t
    abase = wid * A_PER_W
    ebase = t * E_PER_T
    sems = (sem0, sem1)

    pltpu.sync_copy(zeros_hbm, zbuf)

    def idx_load(w, carry):
        pltpu.sync_copy(sidx_hbm.at[pl.ds(abase + w * S_WIN, S_WIN)],
                        idx_all.at[w])
        return carry

    lax.fori_loop(0, S_NWIN, idx_load, 0)

    def zero_acc(j, carry):
        pltpu.sync_copy(zbuf, acc.at[pl.ds(ebase + j * Z_WIN, Z_WIN)])
        return carry

    lax.fori_loop(0, Z_NWIN, zero_acc, 0)
    plsc.subcore_barrier()

    pltpu.sync_copy(ones_hbm, vb2.at[0])

    def cnt_win(w, carry):
        pltpu.sync_copy(vb2.at[0], acc.at[idx_all.at[w]], add=True)
        return carry

    lax.fori_loop(0, S_NWIN, cnt_win, 0)
    plsc.subcore_barrier()

    def cnt_wb(j, carry):
        e0 = ebase + j * Z_WIN
        pltpu.sync_copy(acc.at[pl.ds(e0, Z_WIN)],
                        out_cnt.at[sc, pl.ds(e0, Z_WIN), pl.ds(0, CW)])
        pltpu.sync_copy(zbuf, acc.at[pl.ds(e0, Z_WIN)])
        return carry

    lax.fori_loop(0, Z_NWIN, cnt_wb, 0)
    plsc.subcore_barrier()

    for cix in range(NCHUNK):
        lanes = pl.ds(cix * CW, CW)

        def issue(w, b):
            pltpu.async_copy(
                vals_hbm.at[pl.ds(abase + w * S_WIN, S_WIN), lanes],
                vb2.at[b], sems[b])

        def drain_add(w, b):
            pltpu.make_async_copy(
                vals_hbm.at[pl.ds(abase, S_WIN), lanes],
                vb2.at[b], sems[b]).wait()
            pltpu.sync_copy(vb2.at[b], acc.at[idx_all.at[w]], add=True)

        issue(0, 0)

        def sum_pair(w2, carry2):
            w = 2 * w2 + 1

            @pl.when(w < S_NWIN)
            def _():
                issue(w, 1)

            drain_add(w - 1, 0)

            @pl.when(w + 1 < S_NWIN)
            def _():
                issue(w + 1, 0)

            @pl.when(w < S_NWIN)
            def _():
                drain_add(w, 1)

            return carry2

        lax.fori_loop(0, (S_NWIN + 1) // 2, sum_pair, 0)
        plsc.subcore_barrier()

        def sum_wb(j, carry2):
            e0 = ebase + j * Z_WIN
            pltpu.sync_copy(acc.at[pl.ds(e0, Z_WIN)],
                            out_sum.at[sc, pl.ds(e0, Z_WIN), lanes])
            pltpu.sync_copy(zbuf, acc.at[pl.ds(e0, Z_WIN)])
            return carry2

        lax.fori_loop(0, Z_NWIN, sum_wb, 0)
        plsc.subcore_barrier()


def _k2_body(angle_ref, g0_ref, g1_ref, idx_ref, w_ref, tg_ref, st_ref):
    i = pl.program_id(0)
    wa = w_ref[0:FEAT_A, :]
    w0 = w_ref[FEAT_A:FEAT_A + FEAT, :]
    w1 = w_ref[FEAT_A + FEAT:, :]
    dn = (((0,), (0,)), ((), ()))
    tg = (lax.dot_general(angle_ref[...], wa, dn,
                          preferred_element_type=jnp.float32,
                          precision=HIGH)
          + jnp.dot(g0_ref[...], w0, preferred_element_type=jnp.float32,
                    precision=HIGH)
          + jnp.dot(g1_ref[...], w1, preferred_element_type=jnp.float32,
                    precision=HIGH))
    tg_ref[...] = tg.astype(jnp.bfloat16)
    idxv = idx_ref[0]
    ot = (lax.broadcasted_iota(jnp.int32, (NCRYS, BA), 0) == idxv)
    otb = ot.astype(jnp.bfloat16)
    rhs = jnp.concatenate([tg, tg * tg], axis=1).astype(jnp.bfloat16)
    part = jnp.dot(otb, rhs, preferred_element_type=jnp.float32)
    cnt = jnp.sum(ot.astype(jnp.float32), axis=1, keepdims=True)
    part = jnp.concatenate(
        [part, jnp.broadcast_to(cnt, (NCRYS, FEAT))], axis=1)

    @pl.when(i == 0)
    def _():
        st_ref[...] = part

    @pl.when(i > 0)
    def _():
        st_ref[...] += part


def _norm_params(st, gamma, beta, width):
    ssum = st[:, :width]
    ssq = st[:, width:2 * width]
    cnt = jnp.maximum(st[:, 2 * width:2 * width + 1], 1.0)
    mean = ssum / cnt
    var = ssq / cnt - mean * mean
    scale = gamma * lax.rsqrt(var + EPS)
    shift = beta - mean * scale
    return scale, shift


def _k3_body(tg_ref, idx_ref, st_ref, g1_ref, b1_ref, wm_ref, vals_ref,
             sc_ref, sh_ref):
    i = pl.program_id(0)

    @pl.when(i == 0)
    def _():
        scale, shift = _norm_params(st_ref[...], g1_ref[...], b1_ref[...],
                                    2 * FEAT)
        sc_ref[...] = scale
        sh_ref[...] = shift

    idxv = idx_ref[0]
    ot = (lax.broadcasted_iota(jnp.int32, (NCRYS, BA), 0) == idxv)
    otb = ot.astype(jnp.bfloat16)
    dn = (((0,), (0,)), ((), ()))
    tab = jnp.concatenate(
        [sc_ref[...], sh_ref[...]], axis=1).astype(jnp.bfloat16)
    both = lax.dot_general(otb, tab, dn, preferred_element_type=jnp.float32)
    scale_r = both[:, :2 * FEAT]
    shift_r = both[:, 2 * FEAT:]
    normed = tg_ref[...].astype(jnp.float32) * scale_r + shift_r
    core = jnp.maximum(normed[:, :FEAT], 0.0)
    fh = normed[:, FEAT:]
    filt = jnp.tanh(jnp.sum(fh * wm_ref[...], axis=1, keepdims=True))
    vals_ref[...] = filt * core


def _k5_body(sum_ref, cnt_ref, idx_ref, summed_ref, st_ref):
    i = pl.program_id(0)
    s = sum_ref[0] + sum_ref[1]
    cnt = cnt_ref[0][:, 0:1] + cnt_ref[1][:, 0:1]
    summed = s / jnp.maximum(cnt, 1.0)
    summed_ref[...] = summed
    idxv = idx_ref[0]
    ot = (lax.broadcasted_iota(jnp.int32, (NCRYS, BE), 0) == idxv)
    otb = ot.astype(jnp.bfloat16)
    rhs = jnp.concatenate(
        [summed, summed * summed], axis=1).astype(jnp.bfloat16)
    part = jnp.dot(otb, rhs, preferred_element_type=jnp.float32)
    ccnt = jnp.sum(ot.astype(jnp.float32), axis=1, keepdims=True)
    part = jnp.concatenate(
        [part, jnp.broadcast_to(ccnt, (NCRYS, FEAT))], axis=1)

    @pl.when(i == 0)
    def _():
        st_ref[...] = part

    @pl.when(i > 0)
    def _():
        st_ref[...] += part


def _k6_body(summed_ref, idx_ref, st_ref, g2_ref, b2_ref, edge_ref,
             w10_ref, b10_ref, w20_ref, b20_ref,
             w11_ref, b11_ref, w21_ref, b21_ref,
             out_ref, sc_ref, sh_ref):
    i = pl.program_id(0)

    @pl.when(i == 0)
    def _():
        scale, shift = _norm_params(st_ref[...], g2_ref[...], b2_ref[...],
                                    FEAT)
        sc_ref[...] = scale
        sh_ref[...] = shift

    idxv = idx_ref[0]
    ot = (lax.broadcasted_iota(jnp.int32, (NCRYS, BE), 0) == idxv)
    otb = ot.astype(jnp.bfloat16)
    dn = (((0,), (0,)), ((), ()))
    tab = jnp.concatenate(
        [sc_ref[...], sh_ref[...]], axis=1).astype(jnp.bfloat16)
    both = lax.dot_general(otb, tab, dn, preferred_element_type=jnp.float32)
    scale_r = both[:, :FEAT]
    shift_r = both[:, FEAT:]
    s = summed_ref[...] * scale_r + shift_r
    h = jnp.maximum(jnp.dot(s, w10_ref[...], preferred_element_type=jnp.float32,
                            precision=HIGH) + b10_ref[...], 0.0)
    s = s + jnp.dot(h, w20_ref[...], preferred_element_type=jnp.float32,
                    precision=HIGH) + b20_ref[...]
    h = jnp.maximum(jnp.dot(s, w11_ref[...], preferred_element_type=jnp.float32,
                            precision=HIGH) + b11_ref[...], 0.0)
    s = s + jnp.dot(h, w21_ref[...], preferred_element_type=jnp.float32,
                    precision=HIGH) + b21_ref[...]
    out_ref[...] = ISQRT2 * jnp.maximum(edge_ref[...] + s, 0.0)


def kernel(edge, angle, angle_nbr_idx, crystal_edge_idx, crystal_angle_idx,
           W_full, W_mask, rW1_0, rb1_0, rW2_0, rb2_0, rW1_1, rb1_1, rW2_1,
           rb2_1, g1, b1, g2, b2):
    f32 = jnp.float32
    src_idx = angle_nbr_idx[:, 0].reshape(-1).astype(jnp.int32)
    nbr1_idx = angle_nbr_idx[:, 1].reshape(-1).astype(jnp.int32)
    ca3 = crystal_angle_idx.astype(jnp.int32).reshape(NBA, 1, BA)
    ce3 = crystal_edge_idx.astype(jnp.int32).reshape(NBE, 1, BE)

    k1 = pl.kernel(
        _gather_body,
        out_type=(
            jax.ShapeDtypeStruct((ANGLES, FEAT), f32),
            jax.ShapeDtypeStruct((ANGLES, FEAT), f32),
        ),
        mesh=plsc.VectorSubcoreMesh(core_axis_name="c", subcore_axis_name="s"),
        scratch_types=[
            pltpu.VMEM((2, G_WIN), jnp.int32),
            pltpu.VMEM((2, G_WIN), jnp.int32),
            pltpu.VMEM((2, G_WIN, FEAT), f32),
            pltpu.VMEM((2, G_WIN, FEAT), f32),
            pltpu.SemaphoreType.DMA,
            pltpu.SemaphoreType.DMA,
            pltpu.SemaphoreType.DMA,
            pltpu.SemaphoreType.DMA,
        ],
        compiler_params=pltpu.CompilerParams(use_tc_tiling_on_sc=False),
    )
    g0, g1_rows = k1(edge, src_idx, nbr1_idx)

    tg, stats1 = pl.pallas_call(
        _k2_body,
        grid=(NBA,),
        in_specs=[
            pl.BlockSpec((FEAT_A, BA), lambda i: (0, i)),
            pl.BlockSpec((BA, FEAT), lambda i: (i, 0)),
            pl.BlockSpec((BA, FEAT), lambda i: (i, 0)),
            pl.BlockSpec((1, 1, BA), lambda i: (i, 0, 0)),
            pl.BlockSpec((2 * FEAT + FEAT_A, 2 * FEAT), lambda i: (0, 0)),
        ],
        out_specs=[
            pl.BlockSpec((BA, 2 * FEAT), lambda i: (i, 0)),
            pl.BlockSpec((NCRYS, 5 * FEAT), lambda i: (0, 0)),
        ],
        out_shape=[
            jax.ShapeDtypeStruct((ANGLES, 2 * FEAT), jnp.bfloat16),
            jax.ShapeDtypeStruct((NCRYS, 5 * FEAT), f32),
        ],
    )(jnp.transpose(angle), g0, g1_rows, ca3, W_full)

    vals = pl.pallas_call(
        _k3_body,
        grid=(NBA,),
        in_specs=[
            pl.BlockSpec((BA, 2 * FEAT), lambda i: (i, 0)),
            pl.BlockSpec((1, 1, BA), lambda i: (i, 0, 0)),
            pl.BlockSpec((NCRYS, 5 * FEAT), lambda i: (0, 0)),
            pl.BlockSpec((1, 2 * FEAT), lambda i: (0, 0)),
            pl.BlockSpec((1, 2 * FEAT), lambda i: (0, 0)),
            pl.BlockSpec((1, FEAT), lambda i: (0, 0)),
        ],
        out_specs=pl.BlockSpec((BA, FEAT), lambda i: (i, 0)),
        out_shape=jax.ShapeDtypeStruct((ANGLES, FEAT), f32),
        scratch_shapes=[
            pltpu.VMEM((NCRYS, 2 * FEAT), f32),
            pltpu.VMEM((NCRYS, 2 * FEAT), f32),
        ],
    )(tg, ca3, stats1, g1.reshape(1, -1), b1.reshape(1, -1),
      W_mask.reshape(1, -1))

    ones_buf = jnp.ones((S_WIN, CW), f32)
    zeros_buf = jnp.zeros((Z_WIN, CW), f32)
    k4 = pl.kernel(
        _scatter_body,
        out_type=(
            jax.ShapeDtypeStruct((NSC, EDGES, FEAT), f32),
            jax.ShapeDtypeStruct((NSC, EDGES, FEAT), f32),
        ),
        mesh=plsc.VectorSubcoreMesh(core_axis_name="c", subcore_axis_name="s"),
        scratch_types=[
            pltpu.VMEM((S_NWIN, S_WIN), jnp.int32),
            pltpu.VMEM((2, S_WIN, CW), f32),
            pltpu.VMEM((Z_WIN, CW), f32),
            pltpu.VMEM_SHARED((EDGES, CW), f32),
            pltpu.SemaphoreType.DMA,
            pltpu.SemaphoreType.DMA,
        ],
        compiler_params=pltpu.CompilerParams(use_tc_tiling_on_sc=False),
    )
    sum2, cnt_parts = k4(vals, src_idx, ones_buf, zeros_buf)

    summed, stats2 = pl.pallas_call(
        _k5_body,
        grid=(NBE,),
        in_specs=[
            pl.BlockSpec((NSC, BE, FEAT), lambda i: (0, i, 0)),
            pl.BlockSpec((NSC, BE, FEAT), lambda i: (0, i, 0)),
            pl.BlockSpec((1, 1, BE), lambda i: (i, 0, 0)),
        ],
        out_specs=[
            pl.BlockSpec((BE, FEAT), lambda i: (i, 0)),
            pl.BlockSpec((NCRYS, 3 * FEAT), lambda i: (0, 0)),
        ],
        out_shape=[
            jax.ShapeDtypeStruct((EDGES, FEAT), f32),
            jax.ShapeDtypeStruct((NCRYS, 3 * FEAT), f32),
        ],
    )(sum2, cnt_parts, ce3)

    out = pl.pallas_call(
        _k6_body,
        grid=(NBE,),
        in_specs=[
            pl.BlockSpec((BE, FEAT), lambda i: (i, 0)),
            pl.BlockSpec((1, 1, BE), lambda i: (i, 0, 0)),
            pl.BlockSpec((NCRYS, 3 * FEAT), lambda i: (0, 0)),
            pl.BlockSpec((1, FEAT), lambda i: (0, 0)),
            pl.BlockSpec((1, FEAT), lambda i: (0, 0)),
            pl.BlockSpec((BE, FEAT), lambda i: (i, 0)),
            pl.BlockSpec((FEAT, MIDF), lambda i: (0, 0)),
            pl.BlockSpec((1, MIDF), lambda i: (0, 0)),
            pl.BlockSpec((MIDF, FEAT), lambda i: (0, 0)),
            pl.BlockSpec((1, FEAT), lambda i: (0, 0)),
            pl.BlockSpec((FEAT, MIDF), lambda i: (0, 0)),
            pl.BlockSpec((1, MIDF), lambda i: (0, 0)),
            pl.BlockSpec((MIDF, FEAT), lambda i: (0, 0)),
            pl.BlockSpec((1, FEAT), lambda i: (0, 0)),
        ],
        out_specs=pl.BlockSpec((BE, FEAT), lambda i: (i, 0)),
        out_shape=jax.ShapeDtypeStruct((EDGES, FEAT), f32),
        scratch_shapes=[
            pltpu.VMEM((NCRYS, FEAT), f32),
            pltpu.VMEM((NCRYS, FEAT), f32),
        ],
    )(summed, ce3, stats2, g2.reshape(1, -1), b2.reshape(1, -1), edge,
      rW1_0, rb1_0.reshape(1, -1), rW2_0, rb2_0.reshape(1, -1),
      rW1_1, rb1_1.reshape(1, -1), rW2_1, rb2_1.reshape(1, -1))
    return out
